# Initial kernel scaffold; baseline (speedup 1.0000x reference)
import jax
import jax.numpy as jnp
from jax.experimental import pallas as pl


def kernel(faces, verts_colors, pix_to_face, bary_coords):
    raise NotImplementedError("write your pallas kernel here")


# speedup vs baseline: 4.4197x; 4.4197x over previous
import functools

import jax
import jax.numpy as jnp
from jax import lax
from jax.experimental import pallas as pl
from jax.experimental.pallas import tpu as pltpu
from jax.experimental.pallas import tpu_sc as plsc

V = 100000
F = 200000
B, H, W = 8, 512, 512
N = B * H * W
NC, NS, L = 2, 16, 16
NW = NC * NS
NPW = N // NW
C = 8192
NCHUNK = NPW // C
GB = 128
DEPTH = 8

_mesh = plsc.VectorSubcoreMesh(
    core_axis_name="c", subcore_axis_name="s", num_cores=NC, num_subcores=NS
)


@functools.partial(
    pl.kernel,
    out_type=jax.ShapeDtypeStruct((3 * N,), jnp.float32),
    mesh=_mesh,
    compiler_params=pltpu.CompilerParams(
        needs_layout_passes=False, use_tc_tiling_on_sc=False
    ),
    scratch_types=[
        pltpu.VMEM((C,), jnp.int32),
        pltpu.VMEM((3 * C,), jnp.float32),
        pltpu.VMEM((C,), jnp.int32),
        pltpu.VMEM((C,), jnp.int32),
        pltpu.VMEM((3 * C,), jnp.int32),
        pltpu.VMEM((3 * C,), jnp.float32),
        pltpu.SemaphoreType.DMA,
    ],
)
def _sc_shade(faces_hbm, colors_hbm, pix_hbm, bary_hbm, out_hbm,
              pixv, barf, fvidx, vid, cidx3, outf, sem):
    wid = lax.axis_index("s") * NC + lax.axis_index("c")
    iota = lax.iota(jnp.int32, L)
    iota3 = iota * 3

    def gather_stream(table, idx_buf, dst_buf, nidx):
        def group(g, _):
            o = pl.multiple_of(g * GB * DEPTH, GB)
            cps = [
                pltpu.async_copy(
                    table.at[idx_buf.at[pl.ds(o + d * GB, GB)]],
                    dst_buf.at[pl.ds(o + d * GB, GB)],
                    sem,
                )
                for d in range(DEPTH)
            ]
            for cp in cps:
                cp.wait()
            return _

        lax.fori_loop(0, nidx // (GB * DEPTH), group, None)

    for chunk in range(NCHUNK):
        base = pl.multiple_of(wid * NPW + chunk * C, C)

        pltpu.sync_copy(pix_hbm.at[pl.ds(base, C)], pixv)
        pltpu.sync_copy(bary_hbm.at[pl.ds(3 * base, 3 * C)], barf)

        def argmax_pass(t, _):
            s = pl.multiple_of(t * L, L)
            f = pixv[pl.ds(s, L)]
            pos = 3 * s + iota3
            b0 = plsc.load_gather(barf, [pos])
            b1 = plsc.load_gather(barf, [pos + 1])
            b2 = plsc.load_gather(barf, [pos + 2])
            j = jnp.where(b1 > b0, 1, 0)
            j = jnp.where(b2 > jnp.maximum(b0, b1), 2, j)
            fvidx[pl.ds(s, L)] = f * 3 + j
            return _

        lax.fori_loop(0, C // L, argmax_pass, None)

        gather_stream(faces_hbm, fvidx, vid, C)

        def expand_pass(t, _):
            s = pl.multiple_of(t * L, L)
            c = vid[pl.ds(s, L)] * 3
            pos = 3 * s + iota3
            plsc.store_scatter(cidx3, [pos], c)
            plsc.store_scatter(cidx3, [pos + 1], c + 1)
            plsc.store_scatter(cidx3, [pos + 2], c + 2)
            return _

        lax.fori_loop(0, C // L, expand_pass, None)

        gather_stream(colors_hbm, cidx3, outf, 3 * C)

        pltpu.sync_copy(outf, out_hbm.at[pl.ds(3 * base, 3 * C)])


def kernel(faces, verts_colors, pix_to_face, bary_coords):
    faces_flat = faces.astype(jnp.int32).reshape(3 * F)
    colors_flat = verts_colors.reshape(3 * V)
    pix = pix_to_face.astype(jnp.int32).reshape(N)
    bary = bary_coords.reshape(3 * N)
    out = _sc_shade(faces_flat, colors_flat, pix, bary)
    return out.reshape(B, H, W, 3)
